# baseline (device time: 129297 ns/iter reference)
import jax
import jax.numpy as jnp
from jax import lax
from jax.experimental import pallas as pl
from jax.experimental.pallas import tpu as pltpu

_KY, _VY, _KX, _VX = 0, 1, 2, 3


def kernel(Q, K, V):
    b, s, h, d = Q.shape
    scale = d ** -0.5

    def body(q_ref, k_ref, v_ref, o_ref, kt, vt, k_rx, v_rx,
             send_sems, recv_sems):
        my_x = lax.axis_index("x")
        my_y = lax.axis_index("y")
        partner = (my_x, 1 - my_y)
        xnbr = (1 - my_x, my_y)

        hh = h // 2
        ones_col = jnp.ones((s, 1), jnp.bfloat16)

        def stage_chunk(bi, ho):
            for hi in range(ho * hh, (ho + 1) * hh):
                kt[bi, hi] = k_ref[bi, :, hi, :].astype(jnp.bfloat16)
                vt[bi, hi] = jnp.concatenate(
                    [v_ref[bi, :, hi, :].astype(jnp.bfloat16), ones_col],
                    axis=1,
                )

        def compute_bh(bi, ho):
            for hi in range(ho * hh, (ho + 1) * hh):
                q = (q_ref[bi, :, hi, :] * scale).astype(jnp.bfloat16)
                s_loc = lax.dot_general(
                    q, kt[bi, hi], (((1,), (1,)), ((), ())),
                    preferred_element_type=jnp.float32,
                )
                s_rem = lax.dot_general(
                    q, k_rx[bi, hi], (((1,), (1,)), ((), ())),
                    preferred_element_type=jnp.float32,
                )
                p_loc = jnp.exp(s_loc.astype(jnp.bfloat16))
                p_rem = jnp.exp(s_rem.astype(jnp.bfloat16))
                o_aug = lax.dot_general(
                    p_loc, vt[bi, hi], (((1,), (0,)), ((), ())),
                    preferred_element_type=jnp.float32,
                ) + lax.dot_general(
                    p_rem, v_rx[bi, hi], (((1,), (0,)), ((), ())),
                    preferred_element_type=jnp.float32,
                )
                o_ref[bi, :, hi, :] = o_aug[:, :d] / o_aug[:, d:d + 1]

        barrier_sem = pltpu.get_barrier_semaphore()

        def with_rdy(rdy_x):
            pl.semaphore_signal(
                barrier_sem, inc=1, device_id=partner,
                device_id_type=pl.DeviceIdType.MESH,
            )
            pl.semaphore_signal(
                rdy_x, inc=1, device_id=xnbr,
                device_id_type=pl.DeviceIdType.MESH,
            )

            def run(a, c):
                a_chunks = [(bi, ho) for bi in a for ho in (0, 1)]
                c_chunks = [(bi, ho) for bi in c for ho in (0, 1)]

                def kv_refs(bi, ho):
                    hs = pl.ds(ho * hh, hh)
                    return (kt.at[bi, hs], k_rx.at[bi, hs],
                            vt.at[bi, hs], v_rx.at[bi, hs])

                direct = []
                for j, (bi, ho) in enumerate(a_chunks):
                    stage_chunk(bi, ho)
                    if j == 0:
                        pl.semaphore_wait(barrier_sem, 1)
                    ks, kr, vs, vr = kv_refs(bi, ho)
                    rk = pltpu.make_async_remote_copy(
                        src_ref=ks, dst_ref=kr,
                        send_sem=send_sems.at[_KY, j],
                        recv_sem=recv_sems.at[_KY, j],
                        device_id=partner,
                        device_id_type=pl.DeviceIdType.MESH,
                    )
                    rv = pltpu.make_async_remote_copy(
                        src_ref=vs, dst_ref=vr,
                        send_sem=send_sems.at[_VY, j],
                        recv_sem=recv_sems.at[_VY, j],
                        device_id=partner,
                        device_id_type=pl.DeviceIdType.MESH,
                    )
                    rk.start()
                    rv.start()
                    direct.append((rk, rv))

                for bi, ho in c_chunks:
                    stage_chunk(bi, ho)

                fwds = []
                for j, (bi, ho) in enumerate(a_chunks):
                    _, kr, _, vr = kv_refs(bi, ho)
                    rk, rv = direct[j]
                    rk.wait_recv()
                    if j == 0:
                        pl.semaphore_wait(rdy_x, 1)
                    fk = pltpu.make_async_remote_copy(
                        src_ref=kr, dst_ref=kr,
                        send_sem=send_sems.at[_KX, j],
                        recv_sem=recv_sems.at[_KX, j],
                        device_id=xnbr,
                        device_id_type=pl.DeviceIdType.MESH,
                    )
                    fk.start()
                    rv.wait_recv()
                    fv = pltpu.make_async_remote_copy(
                        src_ref=vr, dst_ref=vr,
                        send_sem=send_sems.at[_VX, j],
                        recv_sem=recv_sems.at[_VX, j],
                        device_id=xnbr,
                        device_id_type=pl.DeviceIdType.MESH,
                    )
                    fv.start()
                    fwds.append((fk, fv))
                    compute_bh(bi, ho)

                for j, (bi, ho) in enumerate(c_chunks):
                    _, kr, _, vr = kv_refs(bi, ho)
                    wk = pltpu.make_async_remote_copy(
                        src_ref=kr, dst_ref=kr,
                        send_sem=send_sems.at[_KX, j],
                        recv_sem=recv_sems.at[_KX, j],
                        device_id=xnbr,
                        device_id_type=pl.DeviceIdType.MESH,
                    )
                    wv = pltpu.make_async_remote_copy(
                        src_ref=vr, dst_ref=vr,
                        send_sem=send_sems.at[_VX, j],
                        recv_sem=recv_sems.at[_VX, j],
                        device_id=xnbr,
                        device_id_type=pl.DeviceIdType.MESH,
                    )
                    wk.wait_recv()
                    wv.wait_recv()
                    compute_bh(bi, ho)

                for (rk, rv), (fk, fv) in zip(direct, fwds):
                    rk.wait_send()
                    rv.wait_send()
                    fk.wait_send()
                    fv.wait_send()

            @pl.when(my_x == 0)
            def _():
                run([0, 1], [2, 3])

            @pl.when(my_x == 1)
            def _():
                run([2, 3], [0, 1])

        pl.run_scoped(with_rdy, rdy_x=pltpu.SemaphoreType.REGULAR)

    return pl.pallas_call(
        body,
        out_shape=jax.ShapeDtypeStruct((b, s, h, d), jnp.float32),
        in_specs=[pl.BlockSpec(memory_space=pltpu.VMEM)] * 3,
        out_specs=pl.BlockSpec(memory_space=pltpu.VMEM),
        scratch_shapes=[
            pltpu.VMEM((b, h, s, d), jnp.bfloat16),
            pltpu.VMEM((b, h, s, d + 1), jnp.bfloat16),
            pltpu.VMEM((b, h, s, d), jnp.bfloat16),
            pltpu.VMEM((b, h, s, d + 1), jnp.bfloat16),
            pltpu.SemaphoreType.DMA((4, 4)),
            pltpu.SemaphoreType.DMA((4, 4)),
        ],
        compiler_params=pltpu.CompilerParams(
            collective_id=0, vmem_limit_bytes=64 * 1024 * 1024,
        ),
    )(Q, K, V)


# device time: 89078 ns/iter; 1.4515x vs baseline; 1.4515x over previous
import jax
import jax.numpy as jnp
from jax import lax
from jax.experimental import pallas as pl
from jax.experimental.pallas import tpu as pltpu

_KY, _VY, _KX, _VX = 0, 1, 2, 3


def kernel(Q, K, V):
    b, s, h, d = Q.shape
    scale = d ** -0.5

    Qt = jnp.transpose(Q * scale, (0, 2, 1, 3)).astype(jnp.bfloat16)
    Kt = jnp.transpose(K, (0, 2, 1, 3)).astype(jnp.bfloat16)
    Vt = jnp.transpose(V, (0, 2, 1, 3)).astype(jnp.bfloat16)

    def body(q_ref, k_ref, v_ref, o_ref, k_rx, v_rx, o_vmem,
             send_sems, recv_sems, out_sems):
        my_x = lax.axis_index("x")
        my_y = lax.axis_index("y")
        partner = (my_x, 1 - my_y)
        xnbr = (1 - my_x, my_y)

        hh = h // 2

        def compute_bh(bi, ho, oj):
            for hi in range(ho * hh, (ho + 1) * hh):
                q = q_ref[bi, hi]
                s_loc = lax.dot_general(
                    q, k_ref[bi, hi], (((1,), (1,)), ((), ())),
                    preferred_element_type=jnp.float32,
                )
                s_rem = lax.dot_general(
                    q, k_rx[bi, hi], (((1,), (1,)), ((), ())),
                    preferred_element_type=jnp.float32,
                )
                p_loc = jnp.exp(s_loc.astype(jnp.bfloat16))
                p_rem = jnp.exp(s_rem.astype(jnp.bfloat16))
                denom = (
                    jnp.sum(p_loc.astype(jnp.float32), axis=1, keepdims=True)
                    + jnp.sum(p_rem.astype(jnp.float32), axis=1, keepdims=True)
                )
                o_num = lax.dot_general(
                    p_loc, v_ref[bi, hi], (((1,), (0,)), ((), ())),
                    preferred_element_type=jnp.float32,
                ) + lax.dot_general(
                    p_rem, v_rx[bi, hi], (((1,), (0,)), ((), ())),
                    preferred_element_type=jnp.float32,
                )
                o_vmem[bi, :, hi, :] = o_num / denom
            hs = pl.ds(ho * hh, hh)
            cp = pltpu.make_async_copy(
                o_vmem.at[bi, :, hs], o_ref.at[bi, :, hs], out_sems.at[oj]
            )
            cp.start()
            return cp

        barrier_sem = pltpu.get_barrier_semaphore()

        def with_rdy(rdy_x):
            pl.semaphore_signal(
                barrier_sem, inc=1, device_id=partner,
                device_id_type=pl.DeviceIdType.MESH,
            )
            pl.semaphore_signal(
                rdy_x, inc=1, device_id=xnbr,
                device_id_type=pl.DeviceIdType.MESH,
            )
            pl.semaphore_wait(barrier_sem, 1)

            def run(a, c):
                a_chunks = [(bi, ho) for bi in a for ho in (0, 1)]
                c_chunks = [(bi, ho) for bi in c for ho in (0, 1)]
                out_cps = []

                def kv_refs(bi, ho):
                    hs = pl.ds(ho * hh, hh)
                    return (k_ref.at[bi, hs], k_rx.at[bi, hs],
                            v_ref.at[bi, hs], v_rx.at[bi, hs])

                direct = []
                for j, (bi, ho) in enumerate(a_chunks):
                    ks, kr, vs, vr = kv_refs(bi, ho)
                    rk = pltpu.make_async_remote_copy(
                        src_ref=ks, dst_ref=kr,
                        send_sem=send_sems.at[_KY, j],
                        recv_sem=recv_sems.at[_KY, j],
                        device_id=partner,
                        device_id_type=pl.DeviceIdType.MESH,
                    )
                    rv = pltpu.make_async_remote_copy(
                        src_ref=vs, dst_ref=vr,
                        send_sem=send_sems.at[_VY, j],
                        recv_sem=recv_sems.at[_VY, j],
                        device_id=partner,
                        device_id_type=pl.DeviceIdType.MESH,
                    )
                    rk.start()
                    rv.start()
                    direct.append((rk, rv))

                fwds = []
                for j, (bi, ho) in enumerate(a_chunks):
                    _, kr, _, vr = kv_refs(bi, ho)
                    rk, rv = direct[j]
                    rk.wait_recv()
                    if j == 0:
                        pl.semaphore_wait(rdy_x, 1)
                    fk = pltpu.make_async_remote_copy(
                        src_ref=kr, dst_ref=kr,
                        send_sem=send_sems.at[_KX, j],
                        recv_sem=recv_sems.at[_KX, j],
                        device_id=xnbr,
                        device_id_type=pl.DeviceIdType.MESH,
                    )
                    fk.start()
                    rv.wait_recv()
                    fv = pltpu.make_async_remote_copy(
                        src_ref=vr, dst_ref=vr,
                        send_sem=send_sems.at[_VX, j],
                        recv_sem=recv_sems.at[_VX, j],
                        device_id=xnbr,
                        device_id_type=pl.DeviceIdType.MESH,
                    )
                    fv.start()
                    fwds.append((fk, fv))
                    out_cps.append(compute_bh(bi, ho, j))

                for j, (bi, ho) in enumerate(c_chunks):
                    _, kr, _, vr = kv_refs(bi, ho)
                    wk = pltpu.make_async_remote_copy(
                        src_ref=kr, dst_ref=kr,
                        send_sem=send_sems.at[_KX, j],
                        recv_sem=recv_sems.at[_KX, j],
                        device_id=xnbr,
                        device_id_type=pl.DeviceIdType.MESH,
                    )
                    wv = pltpu.make_async_remote_copy(
                        src_ref=vr, dst_ref=vr,
                        send_sem=send_sems.at[_VX, j],
                        recv_sem=recv_sems.at[_VX, j],
                        device_id=xnbr,
                        device_id_type=pl.DeviceIdType.MESH,
                    )
                    wk.wait_recv()
                    wv.wait_recv()
                    out_cps.append(compute_bh(bi, ho, 4 + j))

                for (rk, rv), (fk, fv) in zip(direct, fwds):
                    rk.wait_send()
                    rv.wait_send()
                    fk.wait_send()
                    fv.wait_send()
                for cp in out_cps:
                    cp.wait()

            @pl.when(my_x == 0)
            def _():
                run([0, 1], [2, 3])

            @pl.when(my_x == 1)
            def _():
                run([2, 3], [0, 1])

        pl.run_scoped(with_rdy, rdy_x=pltpu.SemaphoreType.REGULAR)

    return pl.pallas_call(
        body,
        out_shape=jax.ShapeDtypeStruct((b, s, h, d), jnp.float32),
        in_specs=[pl.BlockSpec(memory_space=pltpu.VMEM)] * 3,
        out_specs=pl.BlockSpec(memory_space=pl.ANY),
        scratch_shapes=[
            pltpu.VMEM((b, h, s, d), jnp.bfloat16),
            pltpu.VMEM((b, h, s, d), jnp.bfloat16),
            pltpu.VMEM((b, s, h, d), jnp.float32),
            pltpu.SemaphoreType.DMA((4, 4)),
            pltpu.SemaphoreType.DMA((4, 4)),
            pltpu.SemaphoreType.DMA((8,)),
        ],
        compiler_params=pltpu.CompilerParams(
            collective_id=0, vmem_limit_bytes=64 * 1024 * 1024,
        ),
    )(Qt, Kt, Vt)


# device time: 50192 ns/iter; 2.5760x vs baseline; 1.7747x over previous
import jax
import jax.numpy as jnp
from jax import lax
from jax.experimental import pallas as pl
from jax.experimental.pallas import tpu as pltpu

_KY, _VY, _KX, _VX = 0, 1, 2, 3


def kernel(Q, K, V):
    b, s, h, d = Q.shape
    scale = d ** -0.5

    Qp = jnp.transpose(Q, (0, 2, 3, 1))
    Kp = jnp.transpose(K, (0, 2, 3, 1))
    Vp = jnp.transpose(V, (0, 2, 3, 1))

    def body(q_ref, k_ref, v_ref, o_ref, kt, vt, k_rx, v_rx, o_vmem,
             send_sems, recv_sems, out_sems):
        my_x = lax.axis_index("x")
        my_y = lax.axis_index("y")
        partner = (my_x, 1 - my_y)
        xnbr = (1 - my_x, my_y)

        hh = h // 2
        ones_row = jnp.ones((1, s), jnp.bfloat16)
        cdim0 = (((0,), (0,)), ((), ()))
        cdim1 = (((1,), (1,)), ((), ()))

        def stage_chunk(bi, ho):
            hs = pl.ds(ho * hh, hh)
            kt[bi, hs] = k_ref[bi, hs].astype(jnp.bfloat16)
            vt[bi, hs] = v_ref[bi, hs].astype(jnp.bfloat16)

        def compute_bh(bi, ho, oj):
            for hi in range(ho * hh, (ho + 1) * hh):
                q = (q_ref[bi, hi] * scale).astype(jnp.bfloat16)
                s_loc = lax.dot_general(
                    q, kt[bi, hi], cdim0,
                    preferred_element_type=jnp.float32,
                )
                s_rem = lax.dot_general(
                    q, k_rx[bi, hi], cdim0,
                    preferred_element_type=jnp.float32,
                )
                p_loc = jnp.exp(s_loc.astype(jnp.bfloat16))
                p_rem = jnp.exp(s_rem.astype(jnp.bfloat16))
                denom = lax.dot_general(
                    ones_row, p_loc, cdim1,
                    preferred_element_type=jnp.float32,
                ) + lax.dot_general(
                    ones_row, p_rem, cdim1,
                    preferred_element_type=jnp.float32,
                )
                o_t = lax.dot_general(
                    vt[bi, hi], p_loc, cdim1,
                    preferred_element_type=jnp.float32,
                ) + lax.dot_general(
                    v_rx[bi, hi], p_rem, cdim1,
                    preferred_element_type=jnp.float32,
                )
                o_vmem[bi, hi] = o_t / denom
            hs = pl.ds(ho * hh, hh)
            cp = pltpu.make_async_copy(
                o_vmem.at[bi, hs], o_ref.at[bi, hs], out_sems.at[oj]
            )
            cp.start()
            return cp

        barrier_sem = pltpu.get_barrier_semaphore()

        def with_rdy(rdy_x):
            pl.semaphore_signal(
                barrier_sem, inc=1, device_id=partner,
                device_id_type=pl.DeviceIdType.MESH,
            )
            pl.semaphore_signal(
                rdy_x, inc=1, device_id=xnbr,
                device_id_type=pl.DeviceIdType.MESH,
            )

            def run(a, c):
                a_chunks = [(bi, ho) for bi in a for ho in (0, 1)]
                c_chunks = [(bi, ho) for bi in c for ho in (0, 1)]
                out_cps = []

                def kv_refs(bi, ho):
                    hs = pl.ds(ho * hh, hh)
                    return (kt.at[bi, hs], k_rx.at[bi, hs],
                            vt.at[bi, hs], v_rx.at[bi, hs])

                direct = []
                for j, (bi, ho) in enumerate(a_chunks):
                    stage_chunk(bi, ho)
                    if j == 0:
                        pl.semaphore_wait(barrier_sem, 1)
                    ks, kr, vs, vr = kv_refs(bi, ho)
                    rk = pltpu.make_async_remote_copy(
                        src_ref=ks, dst_ref=kr,
                        send_sem=send_sems.at[_KY, j],
                        recv_sem=recv_sems.at[_KY, j],
                        device_id=partner,
                        device_id_type=pl.DeviceIdType.MESH,
                    )
                    rv = pltpu.make_async_remote_copy(
                        src_ref=vs, dst_ref=vr,
                        send_sem=send_sems.at[_VY, j],
                        recv_sem=recv_sems.at[_VY, j],
                        device_id=partner,
                        device_id_type=pl.DeviceIdType.MESH,
                    )
                    rk.start()
                    rv.start()
                    direct.append((rk, rv))

                for bi, ho in c_chunks:
                    stage_chunk(bi, ho)

                fwds = []
                for j, (bi, ho) in enumerate(a_chunks):
                    _, kr, _, vr = kv_refs(bi, ho)
                    rk, rv = direct[j]
                    rk.wait_recv()
                    if j == 0:
                        pl.semaphore_wait(rdy_x, 1)
                    fk = pltpu.make_async_remote_copy(
                        src_ref=kr, dst_ref=kr,
                        send_sem=send_sems.at[_KX, j],
                        recv_sem=recv_sems.at[_KX, j],
                        device_id=xnbr,
                        device_id_type=pl.DeviceIdType.MESH,
                    )
                    fk.start()
                    rv.wait_recv()
                    fv = pltpu.make_async_remote_copy(
                        src_ref=vr, dst_ref=vr,
                        send_sem=send_sems.at[_VX, j],
                        recv_sem=recv_sems.at[_VX, j],
                        device_id=xnbr,
                        device_id_type=pl.DeviceIdType.MESH,
                    )
                    fv.start()
                    fwds.append((fk, fv))
                    out_cps.append(compute_bh(bi, ho, j))

                for j, (bi, ho) in enumerate(c_chunks):
                    _, kr, _, vr = kv_refs(bi, ho)
                    wk = pltpu.make_async_remote_copy(
                        src_ref=kr, dst_ref=kr,
                        send_sem=send_sems.at[_KX, j],
                        recv_sem=recv_sems.at[_KX, j],
                        device_id=xnbr,
                        device_id_type=pl.DeviceIdType.MESH,
                    )
                    wv = pltpu.make_async_remote_copy(
                        src_ref=vr, dst_ref=vr,
                        send_sem=send_sems.at[_VX, j],
                        recv_sem=recv_sems.at[_VX, j],
                        device_id=xnbr,
                        device_id_type=pl.DeviceIdType.MESH,
                    )
                    wk.wait_recv()
                    wv.wait_recv()
                    out_cps.append(compute_bh(bi, ho, 4 + j))

                for (rk, rv), (fk, fv) in zip(direct, fwds):
                    rk.wait_send()
                    rv.wait_send()
                    fk.wait_send()
                    fv.wait_send()
                for cp in out_cps:
                    cp.wait()

            @pl.when(my_x == 0)
            def _():
                run([0, 1], [2, 3])

            @pl.when(my_x == 1)
            def _():
                run([2, 3], [0, 1])

        pl.run_scoped(with_rdy, rdy_x=pltpu.SemaphoreType.REGULAR)

    out = pl.pallas_call(
        body,
        out_shape=jax.ShapeDtypeStruct((b, h, d, s), jnp.float32),
        in_specs=[pl.BlockSpec(memory_space=pltpu.VMEM)] * 3,
        out_specs=pl.BlockSpec(memory_space=pl.ANY),
        scratch_shapes=[
            pltpu.VMEM((b, h, d, s), jnp.bfloat16),
            pltpu.VMEM((b, h, d, s), jnp.bfloat16),
            pltpu.VMEM((b, h, d, s), jnp.bfloat16),
            pltpu.VMEM((b, h, d, s), jnp.bfloat16),
            pltpu.VMEM((b, h, d, s), jnp.float32),
            pltpu.SemaphoreType.DMA((4, 4)),
            pltpu.SemaphoreType.DMA((4, 4)),
            pltpu.SemaphoreType.DMA((8,)),
        ],
        compiler_params=pltpu.CompilerParams(
            collective_id=0, vmem_limit_bytes=64 * 1024 * 1024,
        ),
    )(Qp, Kp, Vp)
    return jnp.transpose(out, (0, 3, 1, 2))
